# baseline (speedup 1.0000x reference)
import functools

import jax
import jax.numpy as jnp
from jax import lax
from jax.experimental import pallas as pl
from jax.experimental.pallas import tpu as pltpu
from jax.experimental.pallas import tpu_sc as plsc

B = 1024
L = 50
D = 64
V = 1000000
NC = 2
NS = 16
NW = NC * NS
PPW = (B * L) // NW
GSZ = 16
NGRP = 104
PPAD = NGRP * GSZ
EW = 1024
VMAIN = 999936
TAILW = 64
MAXC0 = VMAIN - EW
ACCN = 1152
TRASHROW = 1024
ZROWS = 128


@functools.cache
def _get_mesh():
    return plsc.VectorSubcoreMesh(
        core_axis_name="c", subcore_axis_name="s", num_cores=NC, num_subcores=NS
    )


def _scan_body(sv_hbm, se_hbm, xt_hbm, out_hbm,
               sv_v, se_v, slab, tail, stage, tgt_v, zbuf, acc_sh,
               sem1, sem2, sem3):
    sid = lax.axis_index("s")
    cid = lax.axis_index("c")
    wid = sid * NC + cid
    base = 0
    lanes = lax.broadcasted_iota(jnp.int32, (16,), 0)

    c1 = pltpu.async_copy(sv_hbm.at[wid], sv_v.at[...], sem1)
    c2 = pltpu.async_copy(se_hbm.at[wid], se_v.at[...], sem2)
    c3 = pltpu.async_copy(xt_hbm.at[:, pl.ds(VMAIN, TAILW)], tail.at[...], sem3)

    zeros16 = jnp.zeros((16,), jnp.float32)

    @pl.when(sid == 0)
    def _():
        def zrow(r, carry):
            for j in range(4):
                zbuf[r, pl.ds(j * 16, 16)] = zeros16
            return carry

        lax.fori_loop(0, ZROWS, zrow, 0)
        for j in range(ACCN // ZROWS):
            pltpu.sync_copy(
                zbuf.at[...], acc_sh.at[pl.ds(j * ZROWS, ZROWS)]
            )

    plsc.subcore_barrier()
    c1.wait()
    c2.wait()
    c3.wait()

    def group(g, c0):
        sv16 = sv_v[g >> 3, pl.ds((g & 7) * GSZ, GSZ)]
        se16 = se_v[g >> 3, pl.ds((g & 7) * GSZ, GSZ)]

        tmask = sv16 >= VMAIN
        tcols = jnp.clip(sv16 - VMAIN, 0, TAILW - 1)
        remaining = sv16 < VMAIN

        def wcond(carry):
            c0_, rem, tm = carry
            return jnp.any(rem) | jnp.any(tm)

        def wbody(carry):
            c0_, rem, tm = carry
            mv = jnp.min(jnp.where(rem, sv16, jnp.int32(2**30)))
            need = jnp.any(rem) & (mv >= c0_ + EW)
            c0n = jnp.where(
                need, jnp.minimum(mv & ~jnp.int32(127), jnp.int32(MAXC0)), c0_
            )
            c0n = pl.multiple_of(c0n, 128)

            @pl.when(need)
            def _():
                pltpu.sync_copy(
                    xt_hbm.at[:, pl.ds(c0n, EW)], slab.at[...]
                )

            inw = rem & (sv16 >= c0n) & (sv16 < c0n + EW)
            cols = jnp.clip(sv16 - c0n, 0, EW - 1)
            for f in range(D):
                    frow = jnp.full((16,), f, jnp.int32)
                    gv = plsc.load_gather(slab, [frow, cols], mask=inw)
                    tv = plsc.load_gather(tail, [frow, tcols], mask=tm)
                    plsc.store_scatter(
                        stage, [lanes, jnp.full((16,), f, jnp.int32)],
                        gv, mask=inw,
                    )
                    plsc.store_scatter(
                        stage, [lanes, jnp.full((16,), f, jnp.int32)],
                        tv, mask=tm,
                    )
            done = inw | tm
            tgt = jnp.where(done, se16 + base, jnp.int32(base + TRASHROW))
            tgt_v[g & 1, pl.ds(0, 16)] = tgt
            pltpu.sync_copy(
                stage.at[...], acc_sh.at[tgt_v.at[g & 1]], add=True
            )
            return c0n, rem & ~inw, tm & ~tm

        c0, _, _ = lax.while_loop(wcond, wbody, (c0, remaining, tmask))
        return c0

    lax.fori_loop(0, NGRP, group, jnp.int32(-2 * EW))

    plsc.subcore_barrier()

    @pl.when(sid == 0)
    def _():
        pltpu.sync_copy(acc_sh.at[pl.ds(0, B)], out_hbm.at[cid])


@functools.cache
def _get_scan():
    return pl.kernel(
        _scan_body,
        out_type=jax.ShapeDtypeStruct((NC, B, D), jnp.float32),
        mesh=_get_mesh(),
        scratch_types=[
            pltpu.VMEM((PPAD // 128, 128), jnp.int32),
            pltpu.VMEM((PPAD // 128, 128), jnp.int32),
            pltpu.VMEM((D, EW), jnp.float32),
            pltpu.VMEM((D, TAILW), jnp.float32),
            pltpu.VMEM((GSZ, D), jnp.float32),
            pltpu.VMEM((2, 16), jnp.int32),
            pltpu.VMEM((ZROWS, D), jnp.float32),
            pltpu.VMEM_SHARED((ACCN, D), jnp.float32),
            pltpu.SemaphoreType.DMA,
            pltpu.SemaphoreType.DMA,
            pltpu.SemaphoreType.DMA,
        ],
        compiler_params=pltpu.CompilerParams(
            use_tc_tiling_on_sc=False, needs_layout_passes=False
        ),
    )


def _linear_body(acc_ref, wt_ref, b_ref, out_ref):
    a = acc_ref[...]
    his = jnp.sum(a, axis=0)
    out_ref[...] = (
        jnp.dot(his, wt_ref[...], preferred_element_type=jnp.float32)
        + b_ref[...]
    )


def kernel(entities, history, entities_emb, W, b):
    del entities
    flat = history.astype(jnp.int32).reshape(B * L)
    ev = (jnp.arange(B * L, dtype=jnp.int32) // L)
    sv, se = lax.sort((flat, ev), num_keys=1)
    sv = sv.reshape(NW, PPW)
    se = se.reshape(NW, PPW)
    padv = jnp.broadcast_to(sv[:, -1:], (NW, PPAD - PPW))
    sv = jnp.concatenate([sv, padv], axis=1).reshape(NW, PPAD // 128, 128)
    se = jnp.pad(
        se, ((0, 0), (0, PPAD - PPW)), constant_values=TRASHROW
    ).reshape(NW, PPAD // 128, 128)

    acc = _get_scan()(sv, se, entities_emb.T)

    out = pl.pallas_call(
        _linear_body,
        out_shape=jax.ShapeDtypeStruct((B, D), jnp.float32),
    )(acc, W.T, b.reshape(1, D))
    return out


# speedup vs baseline: 7.8716x; 7.8716x over previous
import functools

import jax
import jax.numpy as jnp
from jax import lax
from jax.experimental import pallas as pl
from jax.experimental.pallas import tpu as pltpu
from jax.experimental.pallas import tpu_sc as plsc

B = 1024
L = 50
D = 64
V = 1000000
NC = 2
NS = 16
NW = NC * NS
BPW = B // NW
RPW = BPW * L
CHUNK = 128
NCHUNK = (RPW + CHUNK - 1) // CHUNK
RPAD = NCHUNK * CHUNK
NRING = 4
ACCROWS = 40
TRASH = 32


@functools.cache
def _get_mesh():
    return plsc.VectorSubcoreMesh(
        core_axis_name="c", subcore_axis_name="s", num_cores=NC, num_subcores=NS
    )


def _pool_body(idx_hbm, ev_hbm, x_hbm, out_hbm,
               idx_v, ev_v, r0, r1, r2, r3, acc_sh, isem, esem, gsems, ssems):
    sid = lax.axis_index("s")
    wid = sid * NC + lax.axis_index("c")
    rings = [r0, r1, r2, r3]

    icpy = pltpu.async_copy(idx_hbm.at[wid], idx_v.at[...], isem)
    ecpy = pltpu.async_copy(ev_hbm.at[wid], ev_v.at[...], esem)

    zeros16 = jnp.zeros((16,), jnp.float32)

    def zrow(r, carry):
        for j in range(4):
            r0[r, pl.ds(j * 16, 16)] = zeros16
        return carry

    lax.fori_loop(0, ACCROWS, zrow, 0)
    slab = sid * ACCROWS
    pltpu.sync_copy(r0.at[pl.ds(0, ACCROWS)], acc_sh.at[pl.ds(slab, ACCROWS)])
    icpy.wait()
    ecpy.wait()

    off = jnp.full((16,), slab, jnp.int32)

    def orow(r, carry):
        for j in range(8):
            ev_v[r, pl.ds(j * 16, 16)] = ev_v[r, pl.ds(j * 16, 16)] + off
        return carry

    lax.fori_loop(0, NCHUNK, orow, 0)

    gds = {}
    sds = {}
    for step in range(NCHUNK + 1):
        if step < NCHUNK:
            slot = step % NRING
            if step >= NRING:
                sds[step - NRING].wait()
            gds[step] = pltpu.async_copy(
                x_hbm.at[idx_v.at[step]], rings[slot], gsems.at[slot]
            )
        k = step - 1
        if 0 <= k < NCHUNK:
            gds[k].wait()
            sds[k] = pltpu.async_copy(
                rings[k % NRING], acc_sh.at[ev_v.at[k]], ssems.at[k % NRING],
                add=True,
            )
    for k in range(max(0, NCHUNK - NRING + 1), NCHUNK):
        sds[k].wait()

    pltpu.sync_copy(acc_sh.at[pl.ds(slab, BPW)], out_hbm.at[wid])


@functools.cache
def _get_pool():
    return pl.kernel(
        _pool_body,
        out_type=jax.ShapeDtypeStruct((NW, BPW, D), jnp.float32),
        mesh=_get_mesh(),
        scratch_types=[
            pltpu.VMEM((NCHUNK, CHUNK), jnp.int32),
            pltpu.VMEM((NCHUNK, CHUNK), jnp.int32),
            pltpu.VMEM((CHUNK, D), jnp.float32),
            pltpu.VMEM((CHUNK, D), jnp.float32),
            pltpu.VMEM((CHUNK, D), jnp.float32),
            pltpu.VMEM((CHUNK, D), jnp.float32),
            pltpu.VMEM_SHARED((NS * ACCROWS, D), jnp.float32),
            pltpu.SemaphoreType.DMA,
            pltpu.SemaphoreType.DMA,
            pltpu.SemaphoreType.DMA((NRING,)),
            pltpu.SemaphoreType.DMA((NRING,)),
        ],
        compiler_params=pltpu.CompilerParams(use_tc_tiling_on_sc=False),
    )


def _linear_body(his_ref, wt_ref, b_ref, out_ref):
    out_ref[...] = (
        jnp.dot(
            his_ref[...].reshape(B, D), wt_ref[...],
            preferred_element_type=jnp.float32,
        )
        + b_ref[...]
    )


def kernel(entities, history, entities_emb, W, b):
    del entities
    hist = history.astype(jnp.int32).reshape(NW, RPW)
    e_i = jnp.broadcast_to(
        (jnp.arange(RPW, dtype=jnp.int32) // L)[None, :], (NW, RPW)
    )
    idx = jnp.pad(hist, ((0, 0), (0, RPAD - RPW))).reshape(NW, NCHUNK, CHUNK)
    tgt = jnp.pad(
        e_i, ((0, 0), (0, RPAD - RPW)), constant_values=TRASH
    ).reshape(NW, NCHUNK, CHUNK)

    his = _get_pool()(idx, tgt, entities_emb)

    out = pl.pallas_call(
        _linear_body,
        out_shape=jax.ShapeDtypeStruct((B, D), jnp.float32),
    )(his, W.T, b.reshape(1, D))
    return out
